# baseline (speedup 1.0000x reference)
import functools

import jax
import jax.numpy as jnp
from jax.experimental import pallas as pl
from jax.experimental.pallas import tpu as pltpu


def _vbpr_kernel(nsteps, u_rows, sb_rows,
                 item_ref, vfa_ref, vfb_ref, vfc_ref, vfd_ref,
                 w_ref, b_ref, user_hbm, out_hbm,
                 obuf, ubuf, out_sem, uin_sem, uout_sem):
    i = pl.program_id(0)
    slot = jax.lax.rem(i, 2)
    r = sb_rows // 4
    uc = u_rows // nsteps

    def user_in(step, sl):
        return pltpu.make_async_copy(
            user_hbm.at[pl.ds(step * uc, uc), :], ubuf.at[sl], uin_sem.at[sl])

    def user_out(step, sl):
        return pltpu.make_async_copy(
            ubuf.at[sl], out_hbm.at[pl.ds(step * uc, uc), :], uout_sem.at[sl])

    def block_out(step, sl):
        return pltpu.make_async_copy(
            obuf.at[sl],
            out_hbm.at[pl.ds(u_rows + step * sb_rows, sb_rows), :],
            out_sem.at[sl])

    @pl.when(i >= 2)
    def _():
        user_out(i - 2, slot).wait()
    user_in(i, slot).start()

    @pl.when(i >= 1)
    def _():
        user_in(i - 1, 1 - slot).wait()
        user_out(i - 1, 1 - slot).start()

    @pl.when(i >= 2)
    def _():
        block_out(i - 2, slot).wait()

    dn = (((1,), (1,)), ((), ()))
    for q, vf_ref in enumerate((vfa_ref, vfb_ref, vfc_ref, vfd_ref)):
        vis = jax.lax.dot_general(vf_ref[...], w_ref[...], dn,
                                  preferred_element_type=jnp.float32) + b_ref[...]
        obuf[slot, q * r:(q + 1) * r] = jnp.concatenate(
            [item_ref[q * r:(q + 1) * r], vis], axis=-1)

    block_out(i, slot).start()

    @pl.when(i == nsteps - 1)
    def _():
        user_in(nsteps - 1, slot).wait()
        user_out(nsteps - 1, slot).start()
        user_out(nsteps - 1, slot).wait()
        @pl.when(nsteps >= 2)
        def _():
            user_out(nsteps - 2, 1 - slot).wait()
        for step in range(max(nsteps - 2, 0), nsteps):
            block_out(step, step % 2).wait()


def kernel(user_embedding, item_embedding, v_feat, W, b):
    U, DU = user_embedding.shape
    I, DI = item_embedding.shape
    _, DV = v_feat.shape
    DO = W.shape[0]
    SB = None
    for cand in (4000, 2000, 1000, 400, 80, 32):
        if I % cand == 0 and (cand // 4) % 8 == 0 and U % (I // cand) == 0:
            SB = cand
            break
    ni = I // SB
    b2 = b.reshape(1, DO)

    out = pl.pallas_call(
        functools.partial(_vbpr_kernel, ni, U, SB),
        grid=(ni,),
        in_specs=[
            pl.BlockSpec((SB, DI), lambda i: (i, 0)),
            pl.BlockSpec((SB // 4, DV), lambda i: (4 * i, 0)),
            pl.BlockSpec((SB // 4, DV), lambda i: (4 * i + 1, 0)),
            pl.BlockSpec((SB // 4, DV), lambda i: (4 * i + 2, 0)),
            pl.BlockSpec((SB // 4, DV), lambda i: (4 * i + 3, 0)),
            pl.BlockSpec((DO, DV), lambda i: (0, 0)),
            pl.BlockSpec((1, DO), lambda i: (0, 0)),
            pl.BlockSpec(memory_space=pl.ANY),
        ],
        out_specs=pl.BlockSpec(memory_space=pl.ANY),
        out_shape=jax.ShapeDtypeStruct((U + I, DU), user_embedding.dtype),
        scratch_shapes=[
            pltpu.VMEM((2, SB, DU), jnp.float32),
            pltpu.VMEM((2, U // ni, DU), jnp.float32),
            pltpu.SemaphoreType.DMA((2,)),
            pltpu.SemaphoreType.DMA((2,)),
            pltpu.SemaphoreType.DMA((2,)),
        ],
        compiler_params=pltpu.CompilerParams(
            dimension_semantics=("arbitrary",),
        ),
    )(item_embedding, v_feat, v_feat, v_feat, v_feat, W, b2, user_embedding)
    return out


# speedup vs baseline: 1.0028x; 1.0028x over previous
import functools

import jax
import jax.numpy as jnp
from jax.experimental import pallas as pl
from jax.experimental.pallas import tpu as pltpu


def _vbpr_kernel(nsteps, u_rows, sb_rows,
                 item_ref, vfa_ref, vfb_ref, w_ref, b_ref, user_hbm, out_hbm,
                 obuf, ubuf, out_sem, uin_sem, uout_sem):
    i = pl.program_id(0)
    slot = jax.lax.rem(i, 2)
    r = sb_rows // 2
    uc = u_rows // nsteps

    def user_in(step, sl):
        return pltpu.make_async_copy(
            user_hbm.at[pl.ds(step * uc, uc), :], ubuf.at[sl], uin_sem.at[sl])

    def user_out(step, sl):
        return pltpu.make_async_copy(
            ubuf.at[sl], out_hbm.at[pl.ds(step * uc, uc), :], uout_sem.at[sl])

    def block_out(step, sl):
        return pltpu.make_async_copy(
            obuf.at[sl],
            out_hbm.at[pl.ds(u_rows + step * sb_rows, sb_rows), :],
            out_sem.at[sl])

    @pl.when(i >= 2)
    def _():
        user_out(i - 2, slot).wait()
    user_in(i, slot).start()

    @pl.when(i >= 1)
    def _():
        user_in(i - 1, 1 - slot).wait()
        user_out(i - 1, 1 - slot).start()

    @pl.when(i >= 2)
    def _():
        block_out(i - 2, slot).wait()

    dn = (((1,), (1,)), ((), ()))
    vis_a = jax.lax.dot_general(vfa_ref[...], w_ref[...], dn,
                                preferred_element_type=jnp.float32) + b_ref[...]
    vis_b = jax.lax.dot_general(vfb_ref[...], w_ref[...], dn,
                                preferred_element_type=jnp.float32) + b_ref[...]
    obuf[slot, :r] = jnp.concatenate([item_ref[:r], vis_a], axis=-1)
    obuf[slot, r:] = jnp.concatenate([item_ref[r:], vis_b], axis=-1)

    block_out(i, slot).start()

    @pl.when(i == nsteps - 1)
    def _():
        user_in(nsteps - 1, slot).wait()
        user_out(nsteps - 1, slot).start()
        user_out(nsteps - 1, slot).wait()
        @pl.when(nsteps >= 2)
        def _():
            user_out(nsteps - 2, 1 - slot).wait()
        for step in range(max(nsteps - 2, 0), nsteps):
            block_out(step, step % 2).wait()


def kernel(user_embedding, item_embedding, v_feat, W, b):
    U, DU = user_embedding.shape
    I, DI = item_embedding.shape
    _, DV = v_feat.shape
    DO = W.shape[0]
    SB = None
    for cand in (4000, 2000, 1000, 400, 80, 16):
        if I % cand == 0 and (cand // 2) % 8 == 0 and U % (I // cand) == 0:
            SB = cand
            break
    ni = I // SB
    b2 = b.reshape(1, DO)

    out = pl.pallas_call(
        functools.partial(_vbpr_kernel, ni, U, SB),
        grid=(ni,),
        in_specs=[
            pl.BlockSpec((SB, DI), lambda i: (i, 0)),
            pl.BlockSpec((SB // 2, DV), lambda i: (2 * i, 0)),
            pl.BlockSpec((SB // 2, DV), lambda i: (2 * i + 1, 0)),
            pl.BlockSpec((DO, DV), lambda i: (0, 0)),
            pl.BlockSpec((1, DO), lambda i: (0, 0)),
            pl.BlockSpec(memory_space=pl.ANY),
        ],
        out_specs=pl.BlockSpec(memory_space=pl.ANY),
        out_shape=jax.ShapeDtypeStruct((U + I, DU), user_embedding.dtype),
        scratch_shapes=[
            pltpu.VMEM((2, SB, DU), jnp.float32),
            pltpu.VMEM((2, U // ni, DU), jnp.float32),
            pltpu.SemaphoreType.DMA((2,)),
            pltpu.SemaphoreType.DMA((2,)),
            pltpu.SemaphoreType.DMA((2,)),
        ],
        compiler_params=pltpu.CompilerParams(
            dimension_semantics=("arbitrary",),
        ),
    )(item_embedding, v_feat, v_feat, W, b2, user_embedding)
    return out
